# Initial kernel scaffold; baseline (speedup 1.0000x reference)
import jax
import jax.numpy as jnp
from jax.experimental import pallas as pl


def kernel(warehouse_features, site_features, edge_index, edge_weight):
    raise NotImplementedError("write your pallas kernel here")


# speedup vs baseline: 3.5709x; 3.5709x over previous
import functools

import jax
import jax.numpy as jnp
from jax import lax
from jax.experimental import pallas as pl
from jax.experimental.pallas import tpu as pltpu
from jax.experimental.pallas import tpu_sc as plsc

N_W = 2000
N_S = 8000
N = N_W + N_S
D = 128
E = 320000
NUM_LAYERS = 2

NUM_CORES = 2
NUM_SUBCORES = 16
NUM_TILES = NUM_CORES * NUM_SUBCORES
EDGES_PER_TILE = E // NUM_TILES
CHUNK = 80
NCHUNK = EDGES_PER_TILE // CHUNK
ROWS_MAIN = 624
TAIL = N - NUM_SUBCORES * ROWS_MAIN


def _spmm_body(table_hbm, src_hbm, dst_hbm, w_hbm, out_hbm,
               src_v, dst_v, w_v, rows_v, acc_sh, sem):
    cid = lax.axis_index("c")
    sid = lax.axis_index("s")
    wid = cid * NUM_SUBCORES + sid

    zero = jnp.zeros((16,), jnp.float32)

    def zrow(i, c):
        for r in range(D // 16):
            rows_v[i, pl.ds(r * 16, 16)] = zero
        return c

    lax.fori_loop(0, CHUNK, zrow, 0)
    for k in range(ROWS_MAIN // CHUNK):
        pltpu.sync_copy(
            rows_v, acc_sh.at[pl.ds(sid * ROWS_MAIN + k * CHUNK, CHUNK)])
    pltpu.sync_copy(
        rows_v.at[pl.ds(0, ROWS_MAIN % CHUNK)],
        acc_sh.at[pl.ds(sid * ROWS_MAIN + (ROWS_MAIN // CHUNK) * CHUNK,
                        ROWS_MAIN % CHUNK)])

    @pl.when(sid == NUM_SUBCORES - 1)
    def _zero_tail():
        pltpu.sync_copy(rows_v.at[pl.ds(0, TAIL)],
                        acc_sh.at[pl.ds(NUM_SUBCORES * ROWS_MAIN, TAIL)])

    plsc.subcore_barrier()

    def chunk_body(j, c):
        pltpu.sync_copy(src_hbm.at[wid, j], src_v)
        pltpu.sync_copy(dst_hbm.at[wid, j], dst_v)
        pltpu.sync_copy(w_hbm.at[wid, j], w_v)

        pltpu.async_copy(table_hbm.at[src_v.at[0]], rows_v, sem).wait()

        def group_body(g, cc):
            w16 = w_v[0, pl.ds(g * 16, 16)]
            base = g * 16
            for i in range(16):
                w = w16[i]
                for r in range(D // 16):
                    sl = pl.ds(r * 16, 16)
                    rows_v[base + i, sl] = rows_v[base + i, sl] * w
            return cc

        lax.fori_loop(0, CHUNK // 16, group_body, 0)

        pltpu.sync_copy(rows_v, acc_sh.at[dst_v.at[0]], add=True)
        return c

    lax.fori_loop(0, NCHUNK, chunk_body, 0)
    plsc.subcore_barrier()

    pltpu.sync_copy(acc_sh.at[pl.ds(sid * ROWS_MAIN, ROWS_MAIN)],
                    out_hbm.at[cid, pl.ds(sid * ROWS_MAIN, ROWS_MAIN)])

    @pl.when(sid == NUM_SUBCORES - 1)
    def _copy_tail():
        off = NUM_SUBCORES * ROWS_MAIN
        pltpu.sync_copy(acc_sh.at[pl.ds(off, TAIL)],
                        out_hbm.at[cid, pl.ds(off, TAIL)])


_spmm = functools.partial(
    pl.kernel,
    out_type=jax.ShapeDtypeStruct((NUM_CORES, N, D), jnp.float32),
    mesh=plsc.VectorSubcoreMesh(core_axis_name="c", subcore_axis_name="s"),
    scratch_types=[
        pltpu.VMEM((1, CHUNK), jnp.int32),
        pltpu.VMEM((1, CHUNK), jnp.int32),
        pltpu.VMEM((1, CHUNK), jnp.float32),
        pltpu.VMEM((CHUNK, D), jnp.float32),
        pltpu.VMEM_SHARED((N, D), jnp.float32),
        pltpu.SemaphoreType.DMA,
    ],
)(_spmm_body)


def _combine_body(p0_ref, p1_ref, a_ref, cur_ref, out_ref):
    cur = p0_ref[...] + p1_ref[...]
    ss = jnp.sum(cur * cur, axis=1, keepdims=True)
    norm = jnp.sqrt(ss)
    normed = cur / jnp.maximum(norm, 1e-12)
    cur_ref[...] = cur
    out_ref[...] = a_ref[...] + normed


def _combine(p0, p1, allf):
    blk = 2000
    grid = N // blk
    return pl.pallas_call(
        _combine_body,
        grid=(grid,),
        in_specs=[
            pl.BlockSpec((blk, D), lambda i: (i, 0)),
            pl.BlockSpec((blk, D), lambda i: (i, 0)),
            pl.BlockSpec((blk, D), lambda i: (i, 0)),
        ],
        out_specs=[
            pl.BlockSpec((blk, D), lambda i: (i, 0)),
            pl.BlockSpec((blk, D), lambda i: (i, 0)),
        ],
        out_shape=[
            jax.ShapeDtypeStruct((N, D), jnp.float32),
            jax.ShapeDtypeStruct((N, D), jnp.float32),
        ],
    )(p0, p1, allf)


def kernel(warehouse_features, site_features, edge_index, edge_weight):
    features = jnp.concatenate([warehouse_features, site_features], axis=0)
    src = edge_index[0].reshape(NUM_TILES, NCHUNK, 1, CHUNK)
    dst = edge_index[1].reshape(NUM_TILES, NCHUNK, 1, CHUNK)
    w = edge_weight.reshape(NUM_TILES, NCHUNK, 1, CHUNK)
    def layer(carry, _):
        cur, allf = carry
        p = _spmm(cur, src, dst, w)
        cur2, allf2 = _combine(p[0], p[1], allf)
        return (cur2, allf2), None

    (_, allf), _ = lax.scan(layer, (features, features), None,
                            length=NUM_LAYERS)
    return allf[:N_W], allf[N_W:]
